# baseline (speedup 1.0000x reference)
import dataclasses

import jax
import jax.numpy as jnp
from jax import lax
from jax.experimental import pallas as pl
from jax.experimental.pallas import tpu as pltpu
from jax.experimental.pallas import tpu_sc as plsc

N_NODES = 10000
EPS = 1e-5
NC, NS = 2, 16
NW = NC * NS
CHUNK = 128
N_PAD = 10240
RPT = N_PAD // NS

_SC_PARAMS = dataclasses.replace(pltpu.CompilerParams(),
                                 needs_layout_passes=False)


def _seg_sum_kernel(num_rows: int, feat: int, chunk: int, nbuf: int):
  rpw = num_rows // NW
  cpk = CHUNK // chunk
  rpk = rpw // cpk
  mesh = plsc.VectorSubcoreMesh(
      core_axis_name="c", subcore_axis_name="s", num_cores=NC, num_subcores=NS)
  out_type = jax.ShapeDtypeStruct((NC, N_PAD, feat), jnp.float32)
  scratch = [
      pltpu.VMEM((rpk, CHUNK), jnp.int32),
      [pltpu.VMEM((chunk,), jnp.int32) for _ in range(nbuf)],
      [pltpu.VMEM((chunk,), jnp.int32) for _ in range(nbuf)],
      [pltpu.VMEM((chunk, feat), jnp.float32) for _ in range(nbuf)],
      pltpu.VMEM_SHARED((N_PAD, feat), jnp.float32),
      [pltpu.SemaphoreType.DMA for _ in range(nbuf)],
      [pltpu.SemaphoreType.DMA for _ in range(nbuf)],
  ]
  assert nbuf % cpk == 0 and rpw % nbuf == 0

  def body(pk_hbm, data_hbm, zeros_hbm, p_out, pk, sidx, didx, rows, acc,
           gsem, ssem):
    c = lax.axis_index("c")
    s = lax.axis_index("s")
    wid = s * NC + c
    base = wid * rpk

    pltpu.sync_copy(zeros_hbm.at[pl.ds(s * RPT, RPT)],
                    acc.at[pl.ds(s * RPT, RPT)])
    pltpu.sync_copy(pk_hbm.at[pl.ds(base, rpk)], pk)
    plsc.subcore_barrier()

    def unpack(k, e, b):
      row = k // cpk + e // cpk
      off = (e % cpk) * chunk

      @pl.loop(0, chunk // 16)
      def _(q):
        v = pk[row, pl.ds(off + q * 16, 16)]
        sidx[b][pl.ds(q * 16, 16)] = jnp.bitwise_and(v, 0xFFFF)
        didx[b][pl.ds(q * 16, 16)] = jnp.right_shift(v, 16)

    for b in range(nbuf):
      unpack(0, b, b)
      pltpu.async_copy(data_hbm.at[sidx[b]], rows[b], gsem[b])

    @pl.loop(0, rpw, step=nbuf)
    def _(k):
      for b in range(nbuf):
        pltpu.make_async_copy(data_hbm.at[sidx[b]], rows[b], gsem[b]).wait()
        pltpu.async_copy(rows[b], acc.at[didx[b]], ssem[b], add=True)
      for b in range(nbuf):
        pltpu.make_async_copy(rows[b], acc.at[didx[b]], ssem[b]).wait()

        @pl.when(k + nbuf + b < rpw)
        def _():
          unpack(k, nbuf + b, b)
          pltpu.async_copy(data_hbm.at[sidx[b]], rows[b], gsem[b])

    plsc.subcore_barrier()
    pltpu.sync_copy(acc.at[pl.ds(s * RPT, RPT)],
                    p_out.at[c, pl.ds(s * RPT, RPT)])

  return pl.kernel(body, out_type=out_type, mesh=mesh,
                   scratch_types=scratch, compiler_params=_SC_PARAMS)


def _hist_kernel(num_rows: int, chunk: int):
  rpw = num_rows // NW
  mesh = plsc.VectorSubcoreMesh(
      core_axis_name="c", subcore_axis_name="s", num_cores=NC, num_subcores=NS)
  scratch = [
      pltpu.VMEM((rpw, chunk), jnp.int32),
      pltpu.VMEM((N_PAD,), jnp.float32),
  ]

  def body(dst_hbm, h_out, didx, hist):
    c = lax.axis_index("c")
    s = lax.axis_index("s")
    wid = s * NC + c
    base = wid * rpw

    pltpu.sync_copy(dst_hbm.at[pl.ds(base, rpw)], didx)

    @pl.loop(0, N_PAD // 16)
    def _(i):
      hist[pl.ds(i * 16, 16)] = jnp.zeros((16,), jnp.float32)

    ones16 = jnp.ones((16,), jnp.float32)

    @pl.loop(0, rpw)
    def _(k):
      for q in range(chunk // 16):
        iv = didx[k, pl.ds(q * 16, 16)]
        plsc.addupdate_scatter(hist, [iv], ones16)

    pltpu.sync_copy(hist, h_out.at[pl.ds(wid * N_PAD, N_PAD)])

  return pl.kernel(body,
                   out_type=jax.ShapeDtypeStruct((NW * N_PAD,), jnp.float32),
                   mesh=mesh, scratch_types=scratch,
                   compiler_params=_SC_PARAMS)


def _decode_kernel(num_rows: int, feat: int):
  rpw = num_rows // NW
  mesh = plsc.VectorSubcoreMesh(
      core_axis_name="c", subcore_axis_name="s", num_cores=NC, num_subcores=NS)
  scratch = [
      pltpu.VMEM((rpw, CHUNK), jnp.int32),
      pltpu.VMEM((rpw, CHUNK), jnp.int32),
      [pltpu.VMEM((CHUNK, feat), jnp.float32) for _ in range(2)],
      [pltpu.VMEM((CHUNK, feat), jnp.float32) for _ in range(2)],
      [pltpu.VMEM((CHUNK, 64), jnp.float32) for _ in range(2)],
      [pltpu.SemaphoreType.DMA for _ in range(2)],
      [pltpu.SemaphoreType.DMA for _ in range(2)],
      [pltpu.SemaphoreType.DMA for _ in range(2)],
  ]

  def body(a_hbm, b_hbm, z_hbm, out_hbm, aidx, bidx, arows, brows, prod,
           sema, semb, semo):
    c = lax.axis_index("c")
    s = lax.axis_index("s")
    wid = s * NC + c
    base = wid * rpw

    pltpu.sync_copy(a_hbm.at[pl.ds(base, rpw)], aidx)
    pltpu.sync_copy(b_hbm.at[pl.ds(base, rpw)], bidx)
    for b in range(2):
      pltpu.async_copy(z_hbm.at[aidx.at[b]], arows[b], sema[b])
      pltpu.async_copy(z_hbm.at[bidx.at[b]], brows[b], semb[b])

    @pl.loop(0, rpw, step=2)
    def _(k):
      for b in range(2):
        pltpu.make_async_copy(z_hbm.at[aidx.at[0]], arows[b], sema[b]).wait()
        pltpu.make_async_copy(z_hbm.at[bidx.at[0]], brows[b], semb[b]).wait()

        @pl.when(k >= 2)
        def _():
          pltpu.make_async_copy(
              prod[b], out_hbm.at[pl.ds(0, CHUNK)], semo[b]).wait()

        @pl.loop(0, CHUNK)
        def _(i):
          for q in range(64 // 16):
            sl = pl.ds(q * 16, 16)
            prod[b][i, sl] = arows[b][i, sl] * brows[b][i, sl]

        pltpu.async_copy(prod[b],
                         out_hbm.at[pl.ds((base + k + b) * CHUNK, CHUNK)],
                         semo[b])

        @pl.when(k + 2 + b < rpw)
        def _():
          pltpu.async_copy(z_hbm.at[aidx.at[k + 2 + b]], arows[b], sema[b])
          pltpu.async_copy(z_hbm.at[bidx.at[k + 2 + b]], brows[b], semb[b])

    for b in range(2):
      pltpu.make_async_copy(
          prod[b], out_hbm.at[pl.ds(0, CHUNK)], semo[b]).wait()

  return pl.kernel(
      body,
      out_type=jax.ShapeDtypeStruct((num_rows * CHUNK, 64), jnp.float32),
      mesh=mesh, scratch_types=scratch, compiler_params=_SC_PARAMS)


def _tc_rowsum(prod):
  rows = prod.shape[0]
  blk = 8192

  def body(p_ref, o_ref):
    ones = jnp.ones((64, 1), jnp.float32)
    o_ref[...] = lax.dot_general(p_ref[...], ones, (((1,), (0,)), ((), ())),
                                 preferred_element_type=jnp.float32)

  return pl.pallas_call(
      body,
      grid=(rows // blk,),
      in_specs=[pl.BlockSpec((blk, 64), lambda i: (i, 0))],
      out_specs=pl.BlockSpec((blk, 1), lambda i: (i, 0)),
      out_shape=jax.ShapeDtypeStruct((rows, 1), jnp.float32),
  )(prod)


def _tc_layer1(p, hist, x, W1l_s, W1r_s, bias1, W2r, b2):
  def body(p_ref, h_ref, x_ref, wl_ref, wr_ref, b1_ref, w2r_ref,
           b2_ref, hq_ref, r2_ref, invc_ref):
    ones = jnp.ones((NW, 1), jnp.float32)
    hists = h_ref[:, :N_NODES]
    cnt = lax.dot_general(hists, ones, (((0,), (0,)), ((), ())),
                          preferred_element_type=jnp.float32)
    invc = 1.0 / jnp.maximum(cnt, 1.0)
    psum = p_ref[0, :N_NODES] + p_ref[1, :N_NODES]
    agg = psum * invc
    h = agg @ wl_ref[...].T + x_ref[...] @ wr_ref[...].T + b1_ref[...]
    h = jnp.maximum(h, 0.0)
    hq_ref[...] = h
    r2_ref[...] = h @ w2r_ref[...].T + b2_ref[...]
    invc_ref[...] = invc

  return pl.pallas_call(
      body,
      out_shape=(
          jax.ShapeDtypeStruct((N_NODES, 128), jnp.float32),
          jax.ShapeDtypeStruct((N_NODES, 64), jnp.float32),
          jax.ShapeDtypeStruct((N_NODES, 1), jnp.float32),
      ),
  )(p, hist, x, W1l_s, W1r_s, bias1, W2r, b2)


def _tc_layer2(q, invc, r2, W2l):
  def body(q_ref, invc_ref, r2_ref, w2l_ref, z_ref):
    qsum = q_ref[0, :N_NODES] + q_ref[1, :N_NODES]
    agg = qsum * invc_ref[...]
    z_ref[:, :64] = agg @ w2l_ref[...].T + r2_ref[...]
    z_ref[:, 64:] = jnp.zeros((N_NODES, 64), jnp.float32)

  return pl.pallas_call(
      body,
      out_shape=jax.ShapeDtypeStruct((N_NODES, 128), jnp.float32),
  )(q, invc, r2, W2l)


def kernel(x, edge_index, edge_label_index, W1l, W1r, b1, gamma, beta, rm, rv,
           W2l, W2r, b2):
  E = edge_index.shape[1]
  EL = edge_label_index.shape[1]

  ec = 64
  er = -(-E // (ec * NW * 8)) * NW * 8
  lr = -(-EL // (CHUNK * NW * 8)) * NW * 8

  ei = edge_index.astype(jnp.int32)
  pad_e = er * ec - E
  src = jnp.concatenate([ei[0], jnp.zeros((pad_e,), jnp.int32)])
  dst = jnp.concatenate([ei[1], jnp.full((pad_e,), N_PAD - 1, jnp.int32)])
  pk2d = (src | (dst << 16)).reshape(-1, CHUNK)
  dst2d = dst.reshape(er, ec)

  eli = edge_label_index.astype(jnp.int32)
  pad_l = lr * CHUNK - EL
  a_idx = jnp.concatenate([eli[0], jnp.zeros((pad_l,), jnp.int32)])
  b_idx = jnp.concatenate([eli[1], jnp.zeros((pad_l,), jnp.int32)])
  a2d = a_idx.reshape(lr, CHUNK)
  b2d = b_idx.reshape(lr, CHUNK)

  zq = jnp.zeros((N_PAD, 128), jnp.float32)

  scale = gamma / jnp.sqrt(rv + EPS)
  W1l_s = W1l * scale[:, None]
  W1r_s = W1r * scale[:, None]
  bias1 = ((b1 - rm) * scale + beta)[None, :]

  hist = _hist_kernel(er, ec)(dst2d)
  p = _seg_sum_kernel(er, 128, ec, 4)(pk2d, x, zq)
  hq, r2, invc = _tc_layer1(p, hist.reshape(NW, N_PAD), x, W1l_s, W1r_s, bias1,
                            W2r, b2[None, :])
  q = _seg_sum_kernel(er, 128, ec, 4)(pk2d, hq, zq)
  z = _tc_layer2(q, invc, r2, W2l)
  prod = _decode_kernel(lr, 128)(a2d, b2d, z)
  scores = _tc_rowsum(prod).reshape(-1)
  return scores[:EL]


# speedup vs baseline: 1.0330x; 1.0330x over previous
import dataclasses

import jax
import jax.numpy as jnp
from jax import lax
from jax.experimental import pallas as pl
from jax.experimental.pallas import tpu as pltpu
from jax.experimental.pallas import tpu_sc as plsc

N_NODES = 10000
EPS = 1e-5
NC, NS = 2, 16
NW = NC * NS
CHUNK = 128
N_PAD = 10240
RPT = N_PAD // NS

_SC_PARAMS = dataclasses.replace(pltpu.CompilerParams(),
                                 needs_layout_passes=False)


def _seg_sum_kernel(num_rows: int, feat: int, chunk: int, nbuf: int):
  rpw = num_rows // NW
  cpk = CHUNK // chunk
  rpk = rpw // cpk
  mesh = plsc.VectorSubcoreMesh(
      core_axis_name="c", subcore_axis_name="s", num_cores=NC, num_subcores=NS)
  out_type = jax.ShapeDtypeStruct((NC, N_PAD, feat), jnp.float32)
  scratch = [
      pltpu.VMEM((rpk, CHUNK), jnp.int32),
      [pltpu.VMEM((chunk,), jnp.int32) for _ in range(nbuf)],
      [pltpu.VMEM((chunk,), jnp.int32) for _ in range(nbuf)],
      [pltpu.VMEM((chunk, feat), jnp.float32) for _ in range(nbuf)],
      pltpu.VMEM_SHARED((N_PAD, feat), jnp.float32),
      [pltpu.SemaphoreType.DMA for _ in range(nbuf)],
      [pltpu.SemaphoreType.DMA for _ in range(nbuf)],
  ]
  assert nbuf % cpk == 0 and rpw % nbuf == 0

  def body(pk_hbm, data_hbm, zeros_hbm, p_out, pk, sidx, didx, rows, acc,
           gsem, ssem):
    c = lax.axis_index("c")
    s = lax.axis_index("s")
    wid = s * NC + c
    base = wid * rpk

    pltpu.sync_copy(zeros_hbm.at[pl.ds(s * RPT, RPT)],
                    acc.at[pl.ds(s * RPT, RPT)])
    pltpu.sync_copy(pk_hbm.at[pl.ds(base, rpk)], pk)
    plsc.subcore_barrier()

    def unpack(k, e, b):
      row = k // cpk + e // cpk
      off = (e % cpk) * chunk

      @pl.loop(0, chunk // 16)
      def _(q):
        v = pk[row, pl.ds(off + q * 16, 16)]
        sidx[b][pl.ds(q * 16, 16)] = jnp.bitwise_and(v, 0xFFFF)
        didx[b][pl.ds(q * 16, 16)] = jnp.right_shift(v, 16)

    for b in range(nbuf):
      unpack(0, b, b)
      pltpu.async_copy(data_hbm.at[sidx[b]], rows[b], gsem[b])

    @pl.loop(0, rpw, step=nbuf)
    def _(k):
      for b in range(nbuf):
        pltpu.make_async_copy(data_hbm.at[sidx[b]], rows[b], gsem[b]).wait()
        pltpu.async_copy(rows[b], acc.at[didx[b]], ssem[b], add=True)
      for b in range(nbuf):
        pltpu.make_async_copy(rows[b], acc.at[didx[b]], ssem[b]).wait()

        @pl.when(k + nbuf + b < rpw)
        def _():
          unpack(k, nbuf + b, b)
          pltpu.async_copy(data_hbm.at[sidx[b]], rows[b], gsem[b])

    plsc.subcore_barrier()
    pltpu.sync_copy(acc.at[pl.ds(s * RPT, RPT)],
                    p_out.at[c, pl.ds(s * RPT, RPT)])

  return pl.kernel(body, out_type=out_type, mesh=mesh,
                   scratch_types=scratch, compiler_params=_SC_PARAMS)


def _hist_kernel(num_rows: int, chunk: int):
  rpw = num_rows // NW
  mesh = plsc.VectorSubcoreMesh(
      core_axis_name="c", subcore_axis_name="s", num_cores=NC, num_subcores=NS)
  scratch = [
      pltpu.VMEM((rpw, chunk), jnp.int32),
      pltpu.VMEM((N_PAD,), jnp.float32),
  ]

  def body(dst_hbm, h_out, didx, hist):
    c = lax.axis_index("c")
    s = lax.axis_index("s")
    wid = s * NC + c
    base = wid * rpw

    pltpu.sync_copy(dst_hbm.at[pl.ds(base, rpw)], didx)

    @pl.loop(0, N_PAD // 16)
    def _(i):
      hist[pl.ds(i * 16, 16)] = jnp.zeros((16,), jnp.float32)

    ones16 = jnp.ones((16,), jnp.float32)

    @pl.loop(0, rpw)
    def _(k):
      for q in range(chunk // 16):
        iv = didx[k, pl.ds(q * 16, 16)]
        plsc.addupdate_scatter(hist, [iv], ones16)

    pltpu.sync_copy(hist, h_out.at[pl.ds(wid * N_PAD, N_PAD)])

  return pl.kernel(body,
                   out_type=jax.ShapeDtypeStruct((NW * N_PAD,), jnp.float32),
                   mesh=mesh, scratch_types=scratch,
                   compiler_params=_SC_PARAMS)


def _decode_kernel(num_rows: int, feat: int):
  rpw = num_rows // NW
  mesh = plsc.VectorSubcoreMesh(
      core_axis_name="c", subcore_axis_name="s", num_cores=NC, num_subcores=NS)
  scratch = [
      pltpu.VMEM((rpw, CHUNK), jnp.int32),
      pltpu.VMEM((rpw, CHUNK), jnp.int32),
      [pltpu.VMEM((CHUNK, feat), jnp.float32) for _ in range(2)],
      [pltpu.VMEM((CHUNK, feat), jnp.float32) for _ in range(2)],
      [pltpu.VMEM((CHUNK, 64), jnp.float32) for _ in range(2)],
      [pltpu.SemaphoreType.DMA for _ in range(2)],
      [pltpu.SemaphoreType.DMA for _ in range(2)],
      [pltpu.SemaphoreType.DMA for _ in range(2)],
  ]

  def body(a_hbm, b_hbm, z_hbm, out_hbm, aidx, bidx, arows, brows, prod,
           sema, semb, semo):
    c = lax.axis_index("c")
    s = lax.axis_index("s")
    wid = s * NC + c
    base = wid * rpw

    pltpu.sync_copy(a_hbm.at[pl.ds(base, rpw)], aidx)
    pltpu.sync_copy(b_hbm.at[pl.ds(base, rpw)], bidx)
    for b in range(2):
      pltpu.async_copy(z_hbm.at[aidx.at[b]], arows[b], sema[b])
      pltpu.async_copy(z_hbm.at[bidx.at[b]], brows[b], semb[b])

    @pl.loop(0, rpw, step=2)
    def _(k):
      for b in range(2):
        pltpu.make_async_copy(z_hbm.at[aidx.at[0]], arows[b], sema[b]).wait()
        pltpu.make_async_copy(z_hbm.at[bidx.at[0]], brows[b], semb[b]).wait()

        @pl.when(k >= 2)
        def _():
          pltpu.make_async_copy(
              prod[b], out_hbm.at[pl.ds(0, CHUNK)], semo[b]).wait()

        @pl.loop(0, CHUNK)
        def _(i):
          for q in range(64 // 16):
            sl = pl.ds(q * 16, 16)
            prod[b][i, sl] = arows[b][i, sl] * brows[b][i, sl]

        pltpu.async_copy(prod[b],
                         out_hbm.at[pl.ds((base + k + b) * CHUNK, CHUNK)],
                         semo[b])

        @pl.when(k + 2 + b < rpw)
        def _():
          pltpu.async_copy(z_hbm.at[aidx.at[k + 2 + b]], arows[b], sema[b])
          pltpu.async_copy(z_hbm.at[bidx.at[k + 2 + b]], brows[b], semb[b])

    for b in range(2):
      pltpu.make_async_copy(
          prod[b], out_hbm.at[pl.ds(0, CHUNK)], semo[b]).wait()

  return pl.kernel(
      body,
      out_type=jax.ShapeDtypeStruct((num_rows * CHUNK, 64), jnp.float32),
      mesh=mesh, scratch_types=scratch, compiler_params=_SC_PARAMS)


def _tc_rowsum(prod):
  rows = prod.shape[0]
  blk = 8192

  def body(p_ref, o_ref):
    ones = jnp.ones((64, 1), jnp.float32)
    o_ref[...] = lax.dot_general(p_ref[...], ones, (((1,), (0,)), ((), ())),
                                 preferred_element_type=jnp.float32)

  return pl.pallas_call(
      body,
      grid=(rows // blk,),
      in_specs=[pl.BlockSpec((blk, 64), lambda i: (i, 0))],
      out_specs=pl.BlockSpec((blk, 1), lambda i: (i, 0)),
      out_shape=jax.ShapeDtypeStruct((rows, 1), jnp.float32),
  )(prod)


def _tc_layer1(p, hist, x, W1l_s, W1r_s, bias1, W2r, b2):
  def body(p_ref, h_ref, x_ref, wl_ref, wr_ref, b1_ref, w2r_ref,
           b2_ref, hq_ref, r2_ref, invc_ref):
    ones = jnp.ones((NW, 1), jnp.float32)
    hists = h_ref[:, :N_NODES]
    cnt = lax.dot_general(hists, ones, (((0,), (0,)), ((), ())),
                          preferred_element_type=jnp.float32)
    invc = 1.0 / jnp.maximum(cnt, 1.0)
    psum = p_ref[0, :N_NODES] + p_ref[1, :N_NODES]
    agg = psum * invc
    h = agg @ wl_ref[...].T + x_ref[...] @ wr_ref[...].T + b1_ref[...]
    h = jnp.maximum(h, 0.0)
    hq_ref[...] = h
    r2_ref[...] = h @ w2r_ref[...].T + b2_ref[...]
    invc_ref[...] = invc

  return pl.pallas_call(
      body,
      out_shape=(
          jax.ShapeDtypeStruct((N_NODES, 128), jnp.float32),
          jax.ShapeDtypeStruct((N_NODES, 64), jnp.float32),
          jax.ShapeDtypeStruct((N_NODES, 1), jnp.float32),
      ),
  )(p, hist, x, W1l_s, W1r_s, bias1, W2r, b2)


def _tc_layer2(q, invc, r2, W2l):
  def body(q_ref, invc_ref, r2_ref, w2l_ref, z_ref):
    qsum = q_ref[0, :N_NODES] + q_ref[1, :N_NODES]
    agg = qsum * invc_ref[...]
    z_ref[:, :64] = agg @ w2l_ref[...].T + r2_ref[...]
    z_ref[:, 64:] = jnp.zeros((N_NODES, 64), jnp.float32)

  return pl.pallas_call(
      body,
      out_shape=jax.ShapeDtypeStruct((N_NODES, 128), jnp.float32),
  )(q, invc, r2, W2l)


def kernel(x, edge_index, edge_label_index, W1l, W1r, b1, gamma, beta, rm, rv,
           W2l, W2r, b2):
  E = edge_index.shape[1]
  EL = edge_label_index.shape[1]

  ec = 128
  er = -(-E // (ec * NW * 8)) * NW * 8
  lr = -(-EL // (CHUNK * NW * 8)) * NW * 8

  ei = edge_index.astype(jnp.int32)
  pad_e = er * ec - E
  src = jnp.concatenate([ei[0], jnp.zeros((pad_e,), jnp.int32)])
  dst = jnp.concatenate([ei[1], jnp.full((pad_e,), N_PAD - 1, jnp.int32)])
  pk2d = (src | (dst << 16)).reshape(-1, CHUNK)
  dst2d = dst.reshape(er, ec)

  eli = edge_label_index.astype(jnp.int32)
  pad_l = lr * CHUNK - EL
  a_idx = jnp.concatenate([eli[0], jnp.zeros((pad_l,), jnp.int32)])
  b_idx = jnp.concatenate([eli[1], jnp.zeros((pad_l,), jnp.int32)])
  a2d = a_idx.reshape(lr, CHUNK)
  b2d = b_idx.reshape(lr, CHUNK)

  zq = jnp.zeros((N_PAD, 128), jnp.float32)

  scale = gamma / jnp.sqrt(rv + EPS)
  W1l_s = W1l * scale[:, None]
  W1r_s = W1r * scale[:, None]
  bias1 = ((b1 - rm) * scale + beta)[None, :]

  hist = _hist_kernel(er, ec)(dst2d)
  p = _seg_sum_kernel(er, 128, ec, 2)(pk2d, x, zq)
  hq, r2, invc = _tc_layer1(p, hist.reshape(NW, N_PAD), x, W1l_s, W1r_s, bias1,
                            W2r, b2[None, :])
  q = _seg_sum_kernel(er, 128, ec, 2)(pk2d, hq, zq)
  z = _tc_layer2(q, invc, r2, W2l)
  prod = _decode_kernel(lr, 128)(a2d, b2d, z)
  scores = _tc_rowsum(prod).reshape(-1)
  return scores[:EL]


# speedup vs baseline: 1.1049x; 1.0696x over previous
import dataclasses

import jax
import jax.numpy as jnp
from jax import lax
from jax.experimental import pallas as pl
from jax.experimental.pallas import tpu as pltpu
from jax.experimental.pallas import tpu_sc as plsc

N_NODES = 10000
EPS = 1e-5
NC, NS = 2, 16
NW = NC * NS
CHUNK = 128
N_PAD = 10240
RPT = N_PAD // NS

_SC_PARAMS = dataclasses.replace(pltpu.CompilerParams(),
                                 needs_layout_passes=False)


def _seg_sum_kernel(num_rows: int, feat: int, chunk: int, nbuf: int):
  rpw = num_rows // NW
  cpk = CHUNK // chunk
  rpk = rpw // cpk
  mesh = plsc.VectorSubcoreMesh(
      core_axis_name="c", subcore_axis_name="s", num_cores=NC, num_subcores=NS)
  out_type = jax.ShapeDtypeStruct((NC, N_PAD, feat), jnp.float32)
  scratch = [
      pltpu.VMEM((rpk, CHUNK), jnp.int32),
      [pltpu.VMEM((chunk,), jnp.int32) for _ in range(nbuf)],
      [pltpu.VMEM((chunk,), jnp.int32) for _ in range(nbuf)],
      [pltpu.VMEM((chunk, feat), jnp.float32) for _ in range(nbuf)],
      pltpu.VMEM_SHARED((N_PAD, feat), jnp.float32),
      [pltpu.SemaphoreType.DMA for _ in range(nbuf)],
      [pltpu.SemaphoreType.DMA for _ in range(nbuf)],
  ]
  assert nbuf % cpk == 0 and rpw % nbuf == 0

  def body(pk_hbm, data_hbm, zeros_hbm, p_out, pk, sidx, didx, rows, acc,
           gsem, ssem):
    c = lax.axis_index("c")
    s = lax.axis_index("s")
    wid = s * NC + c
    base = wid * rpk

    pltpu.sync_copy(zeros_hbm.at[pl.ds(s * RPT, RPT)],
                    acc.at[pl.ds(s * RPT, RPT)])
    pltpu.sync_copy(pk_hbm.at[pl.ds(base, rpk)], pk)
    plsc.subcore_barrier()

    def unpack(k, e, b):
      row = k // cpk + e // cpk
      off = (e % cpk) * chunk

      @pl.loop(0, chunk // 16)
      def _(q):
        v = pk[row, pl.ds(off + q * 16, 16)]
        sidx[b][pl.ds(q * 16, 16)] = jnp.bitwise_and(v, 0xFFFF)
        didx[b][pl.ds(q * 16, 16)] = jnp.right_shift(v, 16)

    for b in range(nbuf):
      unpack(0, b, b)
      pltpu.async_copy(data_hbm.at[sidx[b]], rows[b], gsem[b])

    @pl.loop(0, rpw, step=nbuf)
    def _(k):
      for b in range(nbuf):
        pltpu.make_async_copy(data_hbm.at[sidx[b]], rows[b], gsem[b]).wait()
        pltpu.async_copy(rows[b], acc.at[didx[b]], ssem[b], add=True)
      for b in range(nbuf):
        pltpu.make_async_copy(rows[b], acc.at[didx[b]], ssem[b]).wait()

        @pl.when(k + nbuf + b < rpw)
        def _():
          unpack(k, nbuf + b, b)
          pltpu.async_copy(data_hbm.at[sidx[b]], rows[b], gsem[b])

    plsc.subcore_barrier()
    pltpu.sync_copy(acc.at[pl.ds(s * RPT, RPT)],
                    p_out.at[c, pl.ds(s * RPT, RPT)])

  return pl.kernel(body, out_type=out_type, mesh=mesh,
                   scratch_types=scratch, compiler_params=_SC_PARAMS)


def _hist_kernel(num_rows: int, chunk: int):
  rpw = num_rows // NW
  mesh = plsc.VectorSubcoreMesh(
      core_axis_name="c", subcore_axis_name="s", num_cores=NC, num_subcores=NS)
  scratch = [
      pltpu.VMEM((rpw, chunk), jnp.int32),
      pltpu.VMEM((N_PAD,), jnp.float32),
  ]

  def body(dst_hbm, h_out, didx, hist):
    c = lax.axis_index("c")
    s = lax.axis_index("s")
    wid = s * NC + c
    base = wid * rpw

    pltpu.sync_copy(dst_hbm.at[pl.ds(base, rpw)], didx)

    @pl.loop(0, N_PAD // 16)
    def _(i):
      hist[pl.ds(i * 16, 16)] = jnp.zeros((16,), jnp.float32)

    ones16 = jnp.ones((16,), jnp.float32)

    @pl.loop(0, rpw)
    def _(k):
      for q in range(chunk // 16):
        iv = didx[k, pl.ds(q * 16, 16)]
        plsc.addupdate_scatter(hist, [iv], ones16)

    pltpu.sync_copy(hist, h_out.at[pl.ds(wid * N_PAD, N_PAD)])

  return pl.kernel(body,
                   out_type=jax.ShapeDtypeStruct((NW * N_PAD,), jnp.float32),
                   mesh=mesh, scratch_types=scratch,
                   compiler_params=_SC_PARAMS)


def _decode_kernel(num_rows: int, feat: int):
  rpw = num_rows // NW
  mesh = plsc.VectorSubcoreMesh(
      core_axis_name="c", subcore_axis_name="s", num_cores=NC, num_subcores=NS)
  scratch = [
      pltpu.VMEM((rpw, CHUNK), jnp.int32),
      pltpu.VMEM((rpw, CHUNK), jnp.int32),
      [pltpu.VMEM((CHUNK, feat), jnp.float32) for _ in range(2)],
      [pltpu.VMEM((CHUNK, feat), jnp.float32) for _ in range(2)],
      pltpu.VMEM((16 * 16,), jnp.float32),
      [pltpu.VMEM((CHUNK,), jnp.float32) for _ in range(2)],
      [pltpu.SemaphoreType.DMA for _ in range(2)],
      [pltpu.SemaphoreType.DMA for _ in range(2)],
  ]

  def body(a_hbm, b_hbm, z_hbm, out_hbm, aidx, bidx, arows, brows, tbuf, res,
           sema, semb):
    c = lax.axis_index("c")
    s = lax.axis_index("s")
    wid = s * NC + c
    base = wid * rpw

    pltpu.sync_copy(a_hbm.at[pl.ds(base, rpw)], aidx)
    pltpu.sync_copy(b_hbm.at[pl.ds(base, rpw)], bidx)
    for b in range(2):
      pltpu.async_copy(z_hbm.at[aidx.at[b]], arows[b], sema[b])
      pltpu.async_copy(z_hbm.at[bidx.at[b]], brows[b], semb[b])

    col0 = jnp.arange(0, 256, 16, dtype=jnp.int32)

    @pl.loop(0, rpw, step=2)
    def _(k):
      for b in range(2):
        pltpu.make_async_copy(z_hbm.at[aidx.at[0]], arows[b], sema[b]).wait()
        pltpu.make_async_copy(z_hbm.at[bidx.at[0]], brows[b], semb[b]).wait()

        @pl.loop(0, CHUNK // 16)
        def _(g):
          for l in range(16):
            i = g * 16 + l
            p = arows[b][i, pl.ds(0, 16)] * brows[b][i, pl.ds(0, 16)]
            for q in range(1, 64 // 16):
              sl = pl.ds(q * 16, 16)
              p = p + arows[b][i, sl] * brows[b][i, sl]
            tbuf[pl.ds(l * 16, 16)] = p
          acc = plsc.load_gather(tbuf, [col0])
          for j in range(1, 16):
            acc = acc + plsc.load_gather(tbuf, [col0 + j])
          res[b][pl.ds(g * 16, 16)] = acc

        pltpu.sync_copy(res[b], out_hbm.at[pl.ds((base + k + b) * CHUNK,
                                                 CHUNK)])

        @pl.when(k + 2 + b < rpw)
        def _():
          pltpu.async_copy(z_hbm.at[aidx.at[k + 2 + b]], arows[b], sema[b])
          pltpu.async_copy(z_hbm.at[bidx.at[k + 2 + b]], brows[b], semb[b])

  return pl.kernel(
      body,
      out_type=jax.ShapeDtypeStruct((num_rows * CHUNK,), jnp.float32),
      mesh=mesh, scratch_types=scratch, compiler_params=_SC_PARAMS)


def _tc_layer1(p, hist, x, W1l_s, W1r_s, bias1, W2r, b2):
  def body(p_ref, h_ref, x_ref, wl_ref, wr_ref, b1_ref, w2r_ref,
           b2_ref, hq_ref, r2_ref, invc_ref):
    ones = jnp.ones((NW, 1), jnp.float32)
    hists = h_ref[:, :N_NODES]
    cnt = lax.dot_general(hists, ones, (((0,), (0,)), ((), ())),
                          preferred_element_type=jnp.float32)
    invc = 1.0 / jnp.maximum(cnt, 1.0)
    psum = p_ref[0, :N_NODES] + p_ref[1, :N_NODES]
    agg = psum * invc
    h = agg @ wl_ref[...].T + x_ref[...] @ wr_ref[...].T + b1_ref[...]
    h = jnp.maximum(h, 0.0)
    hq_ref[...] = h
    r2_ref[...] = h @ w2r_ref[...].T + b2_ref[...]
    invc_ref[...] = invc

  return pl.pallas_call(
      body,
      out_shape=(
          jax.ShapeDtypeStruct((N_NODES, 128), jnp.float32),
          jax.ShapeDtypeStruct((N_NODES, 64), jnp.float32),
          jax.ShapeDtypeStruct((N_NODES, 1), jnp.float32),
      ),
  )(p, hist, x, W1l_s, W1r_s, bias1, W2r, b2)


def _tc_layer2(q, invc, r2, W2l):
  def body(q_ref, invc_ref, r2_ref, w2l_ref, z_ref):
    qsum = q_ref[0, :N_NODES] + q_ref[1, :N_NODES]
    agg = qsum * invc_ref[...]
    z_ref[:, :64] = agg @ w2l_ref[...].T + r2_ref[...]
    z_ref[:, 64:] = jnp.zeros((N_NODES, 64), jnp.float32)

  return pl.pallas_call(
      body,
      out_shape=jax.ShapeDtypeStruct((N_NODES, 128), jnp.float32),
  )(q, invc, r2, W2l)


def kernel(x, edge_index, edge_label_index, W1l, W1r, b1, gamma, beta, rm, rv,
           W2l, W2r, b2):
  E = edge_index.shape[1]
  EL = edge_label_index.shape[1]

  ec = 64
  er = -(-E // (ec * NW * 8)) * NW * 8
  lr = -(-EL // (CHUNK * NW * 8)) * NW * 8

  ei = edge_index.astype(jnp.int32)
  pad_e = er * ec - E
  src = jnp.concatenate([ei[0], jnp.zeros((pad_e,), jnp.int32)])
  dst = jnp.concatenate([ei[1], jnp.full((pad_e,), N_PAD - 1, jnp.int32)])
  pk2d = (src | (dst << 16)).reshape(-1, CHUNK)
  dst2d = dst.reshape(er, ec)

  eli = edge_label_index.astype(jnp.int32)
  pad_l = lr * CHUNK - EL
  a_idx = jnp.concatenate([eli[0], jnp.zeros((pad_l,), jnp.int32)])
  b_idx = jnp.concatenate([eli[1], jnp.zeros((pad_l,), jnp.int32)])
  a2d = a_idx.reshape(lr, CHUNK)
  b2d = b_idx.reshape(lr, CHUNK)

  zq = jnp.zeros((N_PAD, 128), jnp.float32)

  scale = gamma / jnp.sqrt(rv + EPS)
  W1l_s = W1l * scale[:, None]
  W1r_s = W1r * scale[:, None]
  bias1 = ((b1 - rm) * scale + beta)[None, :]

  hist = _hist_kernel(er, ec)(dst2d)
  p = _seg_sum_kernel(er, 128, ec, 4)(pk2d, x, zq)
  hq, r2, invc = _tc_layer1(p, hist.reshape(NW, N_PAD), x, W1l_s, W1r_s, bias1,
                            W2r, b2[None, :])
  q = _seg_sum_kernel(er, 128, ec, 4)(pk2d, hq, zq)
  z = _tc_layer2(q, invc, r2, W2l)
  scores = _decode_kernel(lr, 128)(a2d, b2d, z)
  return scores[:EL]
